# Initial kernel scaffold; baseline (speedup 1.0000x reference)
import jax
import jax.numpy as jnp
from jax.experimental import pallas as pl


def kernel(x_1st, x_2nd, edge, A_causal, A_trivial, W_node_0, b_node_0, W_node_1, b_node_1, W_r0, b_r0, W_r1, b_r1, W_r2, b_r2):
    raise NotImplementedError("write your pallas kernel here")


# speedup vs baseline: 3.6489x; 3.6489x over previous
import functools

import jax
import jax.numpy as jnp
from jax import lax
from jax.experimental import pallas as pl
from jax.experimental.pallas import tpu as pltpu
from jax.experimental.pallas import tpu_sc as plsc

T = 10000
TP = 10240
M = 160000
D = 128
NS = 16
EPT = M // NS
K = 80
NCHUNK = EPT // K
RPT = TP // NS


def _sc_message_passing(x_cat, nout2, nin, a):
    mesh = plsc.VectorSubcoreMesh(core_axis_name="c", subcore_axis_name="s")

    @functools.partial(
        pl.kernel,
        mesh=mesh,
        out_type=jax.ShapeDtypeStruct((2 * TP, D), jnp.float32),
        scratch_types=[
            pltpu.VMEM_SHARED((TP, D), jnp.float32),
            pltpu.VMEM((K,), jnp.int32),
            pltpu.VMEM((K,), jnp.int32),
            pltpu.VMEM((K,), jnp.float32),
            pltpu.VMEM((K, D), jnp.float32),
            pltpu.SemaphoreType.DMA,
        ],
    )
    def k(x_hbm, nout_hbm, nin_hbm, a_hbm, out_hbm,
          acc, idx_out, idx_in, a_v, rows, sem):
        c = lax.axis_index("c")
        s = lax.axis_index("s")
        pltpu.sync_copy(x_hbm.at[pl.ds(c * TP + s * RPT, RPT)],
                        acc.at[pl.ds(s * RPT, RPT)])
        plsc.subcore_barrier()

        def chunk_body(i, carry):
            base = s * EPT + i * K
            pltpu.sync_copy(nout_hbm.at[pl.ds(c * M + base, K)], idx_out)
            pltpu.sync_copy(nin_hbm.at[pl.ds(base, K)], idx_in)
            pltpu.sync_copy(a_hbm.at[pl.ds(base, K)], a_v)
            pltpu.async_copy(x_hbm.at[idx_out], rows, sem).wait()

            def emul(g, carry2):
                ae16 = a_v[pl.ds(g * 16, 16)]
                for j in range(16):
                    e = g * 16 + j
                    aj = jnp.full((16,), ae16[j], jnp.float32)
                    for dd in range(D // 16):
                        sl = pl.ds(dd * 16, 16)
                        rows[e, sl] = rows[e, sl] * aj
                return carry2

            lax.fori_loop(0, K // 16, emul, 0)
            pltpu.sync_copy(rows, acc.at[idx_in], add=True)
            return carry

        lax.fori_loop(0, NCHUNK, chunk_body, 0)

        plsc.subcore_barrier()
        pltpu.sync_copy(acc.at[pl.ds(s * RPT, RPT)],
                        out_hbm.at[pl.ds(c * TP + s * RPT, RPT)])

    return k(x_cat, nout2, nin, a)


def _tc_linear(x, wt, b, relu):
    def body(x_ref, w_ref, b_ref, o_ref):
        y = jnp.dot(x_ref[...], w_ref[...],
                    preferred_element_type=jnp.float32) + b_ref[...]
        if relu:
            y = jnp.maximum(y, 0.0)
        o_ref[...] = y

    n, din = x.shape
    dout = wt.shape[1]
    br = 2048
    assert n % br == 0
    return pl.pallas_call(
        body,
        grid=(n // br,),
        in_specs=[
            pl.BlockSpec((br, din), lambda i: (i, 0)),
            pl.BlockSpec((din, dout), lambda i: (0, 0)),
            pl.BlockSpec((1, dout), lambda i: (0, 0)),
        ],
        out_specs=pl.BlockSpec((br, dout), lambda i: (i, 0)),
        out_shape=jax.ShapeDtypeStruct((n, dout), jnp.float32),
    )(x, wt, b)


def _tc_readout(h, w0t, b0, w1t, b1, w2t, b2):
    def body(h_ref, w0_ref, b0_ref, w1_ref, b1_ref, w2_ref, b2_ref, o_ref):
        hh = jnp.dot(h_ref[...], w0_ref[...],
                     preferred_element_type=jnp.float32) + b0_ref[...]
        hh = jnp.maximum(hh, 0.0)
        hh = jnp.dot(hh, w1_ref[...],
                     preferred_element_type=jnp.float32) + b1_ref[...]
        hh = jnp.maximum(hh, 0.0)
        o_ref[...] = jnp.dot(hh, w2_ref[...],
                             preferred_element_type=jnp.float32) + b2_ref[...]

    n = h.shape[0]
    br = 2000
    assert n % br == 0
    return pl.pallas_call(
        body,
        grid=(n // br,),
        in_specs=[
            pl.BlockSpec((br, 2 * D), lambda i: (i, 0)),
            pl.BlockSpec((2 * D, 2 * D), lambda i: (0, 0)),
            pl.BlockSpec((1, 2 * D), lambda i: (0, 0)),
            pl.BlockSpec((2 * D, D), lambda i: (0, 0)),
            pl.BlockSpec((1, D), lambda i: (0, 0)),
            pl.BlockSpec((D, 1), lambda i: (0, 0)),
            pl.BlockSpec((1, 1), lambda i: (0, 0)),
        ],
        out_specs=pl.BlockSpec((br, 1), lambda i: (i, 0)),
        out_shape=jax.ShapeDtypeStruct((n, 1), jnp.float32),
    )(h, w0t, b0, w1t, b1, w2t, b2)


def kernel(x_1st, x_2nd, edge, A_causal, A_trivial,
           W_node_0, b_node_0, W_node_1, b_node_1,
           W_r0, b_r0, W_r1, b_r1, W_r2, b_r2):
    xc = jnp.transpose(x_1st[0], (1, 0, 2))
    x0 = jnp.pad(xc, ((0, 0), (0, TP - T), (0, 0))).reshape(2 * TP, D)
    nin = edge[0]
    nout = edge[1]
    nout2 = jnp.concatenate([nout, nout + TP])
    a = A_causal.reshape(M)

    m0 = _sc_message_passing(x0, nout2, nin, a)
    x1 = _tc_linear(m0, W_node_0.T, b_node_0.reshape(1, D), relu=True)
    m1 = _sc_message_passing(x1, nout2, nin, a)
    x2 = _tc_linear(m1, W_node_1.T, b_node_1.reshape(1, D), relu=False)

    h = jnp.concatenate([x2[:T], x2[TP:TP + T]], axis=1)
    o = _tc_readout(h, W_r0.T, b_r0.reshape(1, 2 * D),
                    W_r1.T, b_r1.reshape(1, D),
                    W_r2.T, b_r2.reshape(1, 1))
    return o.reshape(1, T, 1)


# speedup vs baseline: 4.8328x; 1.3244x over previous
import functools

import jax
import jax.numpy as jnp
from jax import lax
from jax.experimental import pallas as pl
from jax.experimental.pallas import tpu as pltpu
from jax.experimental.pallas import tpu_sc as plsc

T = 10000
TP = 10240
M = 160000
D = 128
NS = 16
EPT = M // NS
K = 80
EPTP = 10240
NCHUNK = EPTP // K
RPT = TP // NS


def _sc_message_passing(x_cat, packed, a):
    mesh = plsc.VectorSubcoreMesh(core_axis_name="c", subcore_axis_name="s")

    @functools.partial(
        pl.kernel,
        mesh=mesh,
        out_type=jax.ShapeDtypeStruct((2 * TP, D), jnp.float32),
        scratch_types=[
            pltpu.VMEM_SHARED((TP, D), jnp.float32),
            pltpu.VMEM((NCHUNK, K), jnp.int32),
            pltpu.VMEM((K, D), jnp.float32),
            pltpu.VMEM((K, D), jnp.float32),
            pltpu.VMEM((K,), jnp.float32),
            pltpu.VMEM((K,), jnp.float32),
            pltpu.VMEM((K,), jnp.int32),
            pltpu.VMEM((K,), jnp.int32),
            pltpu.VMEM((K,), jnp.int32),
            pltpu.VMEM((K,), jnp.int32),
            pltpu.SemaphoreType.DMA,
            pltpu.SemaphoreType.DMA,
            pltpu.SemaphoreType.DMA,
            pltpu.SemaphoreType.DMA,
        ],
    )
    def k(x_hbm, packed_hbm, a_hbm, out_hbm,
          acc, pk_v, rows0, rows1, ab0, ab1, gi0, gi1, si0, si1,
          gsem0, gsem1, ssem0, ssem1):
        c = lax.axis_index("c")
        s = lax.axis_index("s")
        rows = (rows0, rows1)
        ab = (ab0, ab1)
        gidx = (gi0, gi1)
        sidx = (si0, si1)
        gsem = (gsem0, gsem1)
        ssem = (ssem0, ssem1)

        pltpu.sync_copy(packed_hbm.at[pl.ds((c * NS + s) * NCHUNK, NCHUNK)],
                        pk_v)
        pltpu.sync_copy(x_hbm.at[pl.ds(c * TP + s * RPT, RPT)],
                        acc.at[pl.ds(s * RPT, RPT)])

        def unpack_idx(i, b):
            for v in range(K // 16):
                sl = pl.ds(v * 16, 16)
                pk = pk_v[i, sl]
                gidx[b][sl] = jnp.bitwise_and(pk, 0x7FFF)
                sidx[b][sl] = jnp.right_shift(pk, 15)

        def mul_rows(rows_ref, ab_ref):
            def emul(g, carry):
                ae16 = ab_ref[pl.ds(g * 16, 16)]
                for j in range(16):
                    e = g * 16 + j
                    aj = jnp.full((16,), ae16[j], jnp.float32)
                    for dd in range(D // 16):
                        sl = pl.ds(dd * 16, 16)
                        rows_ref[e, sl] = rows_ref[e, sl] * aj
                return carry

            lax.fori_loop(0, K // 16, emul, 0)

        def gather_start(i, b):
            pltpu.async_copy(x_hbm.at[gidx[b]], rows[b], gsem[b])
            pltpu.async_copy(a_hbm.at[s * NCHUNK + i], ab[b], gsem[b])

        def gather_wait(i, b):
            pltpu.make_async_copy(x_hbm.at[gidx[b]], rows[b], gsem[b]).wait()
            pltpu.make_async_copy(a_hbm.at[s * NCHUNK + i], ab[b],
                                  gsem[b]).wait()

        def scatter_start(b):
            pltpu.async_copy(rows[b], acc.at[sidx[b]], ssem[b], add=True)

        def scatter_wait(b):
            pltpu.make_async_copy(rows[b], acc.at[sidx[b]], ssem[b]).wait()

        plsc.subcore_barrier()

        unpack_idx(0, 0)
        gather_start(0, 0)
        gather_wait(0, 0)
        unpack_idx(1, 1)
        gather_start(1, 1)
        mul_rows(rows[0], ab[0])
        scatter_start(0)

        def pipe_body(g, carry):
            for b in (0, 1):
                i = 2 * g + 1 + b
                beta = (1 + b) % 2
                nb = 1 - beta
                gather_wait(i, beta)
                scatter_wait(nb)
                unpack_idx(i + 1, nb)
                gather_start(i + 1, nb)
                mul_rows(rows[beta], ab[beta])
                scatter_start(beta)
            return carry

        lax.fori_loop(0, (NCHUNK - 2) // 2, pipe_body, 0)

        gather_wait(NCHUNK - 1, 1)
        scatter_wait(0)
        mul_rows(rows[1], ab[1])
        scatter_start(1)
        scatter_wait(1)

        plsc.subcore_barrier()
        pltpu.sync_copy(acc.at[pl.ds(s * RPT, RPT)],
                        out_hbm.at[pl.ds(c * TP + s * RPT, RPT)])

    return k(x_cat, packed, a)


def _tc_linear(x, wt, b, relu):
    def body(x_ref, w_ref, b_ref, o_ref):
        y = jnp.dot(x_ref[...], w_ref[...],
                    preferred_element_type=jnp.float32) + b_ref[...]
        if relu:
            y = jnp.maximum(y, 0.0)
        o_ref[...] = y

    n, din = x.shape
    dout = wt.shape[1]
    br = 2048
    assert n % br == 0
    return pl.pallas_call(
        body,
        grid=(n // br,),
        in_specs=[
            pl.BlockSpec((br, din), lambda i: (i, 0)),
            pl.BlockSpec((din, dout), lambda i: (0, 0)),
            pl.BlockSpec((1, dout), lambda i: (0, 0)),
        ],
        out_specs=pl.BlockSpec((br, dout), lambda i: (i, 0)),
        out_shape=jax.ShapeDtypeStruct((n, dout), jnp.float32),
    )(x, wt, b)


def _tc_readout(h, w0t, b0, w1t, b1, w2t, b2):
    def body(h_ref, w0_ref, b0_ref, w1_ref, b1_ref, w2_ref, b2_ref, o_ref):
        hh = jnp.dot(h_ref[...], w0_ref[...],
                     preferred_element_type=jnp.float32) + b0_ref[...]
        hh = jnp.maximum(hh, 0.0)
        hh = jnp.dot(hh, w1_ref[...],
                     preferred_element_type=jnp.float32) + b1_ref[...]
        hh = jnp.maximum(hh, 0.0)
        o_ref[...] = jnp.dot(hh, w2_ref[...],
                             preferred_element_type=jnp.float32) + b2_ref[...]

    n = h.shape[0]
    br = 2000
    assert n % br == 0
    return pl.pallas_call(
        body,
        grid=(n // br,),
        in_specs=[
            pl.BlockSpec((br, 2 * D), lambda i: (i, 0)),
            pl.BlockSpec((2 * D, 2 * D), lambda i: (0, 0)),
            pl.BlockSpec((1, 2 * D), lambda i: (0, 0)),
            pl.BlockSpec((2 * D, D), lambda i: (0, 0)),
            pl.BlockSpec((1, D), lambda i: (0, 0)),
            pl.BlockSpec((D, 1), lambda i: (0, 0)),
            pl.BlockSpec((1, 1), lambda i: (0, 0)),
        ],
        out_specs=pl.BlockSpec((br, 1), lambda i: (i, 0)),
        out_shape=jax.ShapeDtypeStruct((n, 1), jnp.float32),
    )(h, w0t, b0, w1t, b1, w2t, b2)


def kernel(x_1st, x_2nd, edge, A_causal, A_trivial,
           W_node_0, b_node_0, W_node_1, b_node_1,
           W_r0, b_r0, W_r1, b_r1, W_r2, b_r2):
    xc = jnp.transpose(x_1st[0], (1, 0, 2))
    x0 = jnp.pad(xc, ((0, 0), (0, TP - T), (0, 0))).reshape(2 * TP, D)
    pad = ((0, 0), (0, EPTP - EPT))
    nin_p = jnp.pad(edge[0].reshape(NS, EPT), pad)
    nout_p = jnp.pad(edge[1].reshape(NS, EPT), pad)
    pk0 = jnp.bitwise_or(nout_p, jnp.left_shift(nin_p, 15))
    pk1 = jnp.bitwise_or(nout_p + TP, jnp.left_shift(nin_p, 15))
    packed = jnp.concatenate([pk0, pk1]).reshape(2 * NS * NCHUNK, K)
    a = jnp.pad(A_causal.reshape(NS, EPT), pad).reshape(NS * NCHUNK, K)

    m0 = _sc_message_passing(x0, packed, a)
    x1 = _tc_linear(m0, W_node_0.T, b_node_0.reshape(1, D), relu=True)
    m1 = _sc_message_passing(x1, packed, a)
    x2 = _tc_linear(m1, W_node_1.T, b_node_1.reshape(1, D), relu=False)

    h = jnp.concatenate([x2[:T], x2[TP:TP + T]], axis=1)
    o = _tc_readout(h, W_r0.T, b_r0.reshape(1, 2 * D),
                    W_r1.T, b_r1.reshape(1, D),
                    W_r2.T, b_r2.reshape(1, 1))
    return o.reshape(1, T, 1)


# speedup vs baseline: 5.2206x; 1.0803x over previous
import functools

import jax
import jax.numpy as jnp
from jax import lax
from jax.experimental import pallas as pl
from jax.experimental.pallas import tpu as pltpu
from jax.experimental.pallas import tpu_sc as plsc

T = 10000
TP = 10240
M = 160000
D = 128
NS = 16
EPT = M // NS
K = 64
EPTP = 10240
NCHUNK = EPTP // K
NBUF = 3
RPT = TP // NS


def _sc_message_passing(x_cat, packed, a):
    mesh = plsc.VectorSubcoreMesh(core_axis_name="c", subcore_axis_name="s")

    @functools.partial(
        pl.kernel,
        mesh=mesh,
        out_type=jax.ShapeDtypeStruct((2 * TP, D), jnp.float32),
        scratch_types=[
            pltpu.VMEM_SHARED((TP, D), jnp.float32),
            pltpu.VMEM((NCHUNK, K), jnp.int32),
            pltpu.VMEM((NBUF, K, D), jnp.float32),
            pltpu.VMEM((NBUF, K), jnp.float32),
            pltpu.VMEM((NBUF, K), jnp.int32),
            pltpu.VMEM((NBUF, K), jnp.int32),
            pltpu.SemaphoreType.DMA((NBUF,)),
            pltpu.SemaphoreType.DMA((NBUF,)),
        ],
    )
    def k(x_hbm, packed_hbm, a_hbm, out_hbm,
          acc, pk_v, rowsb, abb, gib, sib, gsem, ssem):
        c = lax.axis_index("c")
        s = lax.axis_index("s")
        rows = tuple(rowsb.at[b] for b in range(NBUF))
        ab = tuple(abb.at[b] for b in range(NBUF))
        gidx = tuple(gib.at[b] for b in range(NBUF))
        sidx = tuple(sib.at[b] for b in range(NBUF))

        pltpu.sync_copy(packed_hbm.at[pl.ds((c * NS + s) * NCHUNK, NCHUNK)],
                        pk_v)
        pltpu.sync_copy(x_hbm.at[pl.ds(c * TP + s * RPT, RPT)],
                        acc.at[pl.ds(s * RPT, RPT)])

        def unpack_idx(i, b):
            for v in range(K // 16):
                sl = pl.ds(v * 16, 16)
                pk = pk_v[i, sl]
                gidx[b][sl] = jnp.bitwise_and(pk, 0x7FFF)
                sidx[b][sl] = jnp.right_shift(pk, 15)

        def mul_rows(b):
            def emul(g, carry):
                ae16 = ab[b][pl.ds(g * 16, 16)]
                for j in range(16):
                    e = g * 16 + j
                    aj = jnp.full((16,), ae16[j], jnp.float32)
                    for dd in range(D // 16):
                        sl = pl.ds(dd * 16, 16)
                        rows[b][e, sl] = rows[b][e, sl] * aj
                return carry

            lax.fori_loop(0, K // 16, emul, 0)

        def gather_start(i, b):
            pltpu.async_copy(x_hbm.at[gidx[b]], rows[b], gsem.at[b])
            pltpu.async_copy(a_hbm.at[s * NCHUNK + i], ab[b], gsem.at[b])

        def gather_wait(i, b):
            pltpu.make_async_copy(x_hbm.at[gidx[b]], rows[b],
                                  gsem.at[b]).wait()
            pltpu.make_async_copy(a_hbm.at[s * NCHUNK + i], ab[b],
                                  gsem.at[b]).wait()

        def scatter_start(b):
            pltpu.async_copy(rows[b], acc.at[sidx[b]], ssem.at[b], add=True)

        def scatter_wait(b):
            pltpu.make_async_copy(rows[b], acc.at[sidx[b]], ssem.at[b]).wait()

        plsc.subcore_barrier()

        def step(i, b, bg, swait, gstart):
            gather_wait(i, b)
            if swait:
                scatter_wait(bg)
            if gstart:
                unpack_idx(i + 2, bg)
                gather_start(i + 2, bg)
            mul_rows(b)
            scatter_start(b)

        unpack_idx(0, 0)
        gather_start(0, 0)
        unpack_idx(1, 1)
        gather_start(1, 1)
        step(0, 0, 2, False, True)

        def pipe_body(g, carry):
            for b3 in (0, 1, 2):
                i = 3 * g + 1 + b3
                step(i, (1 + b3) % 3, b3, True, True)
            return carry

        lax.fori_loop(0, (NCHUNK - 4) // 3, pipe_body, 0)

        step(NCHUNK - 3, 1, 0, True, True)
        step(NCHUNK - 2, 2, 1, True, False)
        step(NCHUNK - 1, 0, 2, True, False)
        scatter_wait(0)

        plsc.subcore_barrier()
        pltpu.sync_copy(acc.at[pl.ds(s * RPT, RPT)],
                        out_hbm.at[pl.ds(c * TP + s * RPT, RPT)])

    return k(x_cat, packed, a)


def _tc_linear(x, wt, b, relu):
    def body(x_ref, w_ref, b_ref, o_ref):
        y = jnp.dot(x_ref[...], w_ref[...],
                    preferred_element_type=jnp.float32) + b_ref[...]
        if relu:
            y = jnp.maximum(y, 0.0)
        o_ref[...] = y

    n, din = x.shape
    dout = wt.shape[1]
    br = 2048
    assert n % br == 0
    return pl.pallas_call(
        body,
        grid=(n // br,),
        in_specs=[
            pl.BlockSpec((br, din), lambda i: (i, 0)),
            pl.BlockSpec((din, dout), lambda i: (0, 0)),
            pl.BlockSpec((1, dout), lambda i: (0, 0)),
        ],
        out_specs=pl.BlockSpec((br, dout), lambda i: (i, 0)),
        out_shape=jax.ShapeDtypeStruct((n, dout), jnp.float32),
    )(x, wt, b)


def _tc_readout(h, w0t, b0, w1t, b1, w2t, b2):
    def body(h_ref, w0_ref, b0_ref, w1_ref, b1_ref, w2_ref, b2_ref, o_ref):
        hh = jnp.dot(h_ref[...], w0_ref[...],
                     preferred_element_type=jnp.float32) + b0_ref[...]
        hh = jnp.maximum(hh, 0.0)
        hh = jnp.dot(hh, w1_ref[...],
                     preferred_element_type=jnp.float32) + b1_ref[...]
        hh = jnp.maximum(hh, 0.0)
        o_ref[...] = jnp.dot(hh, w2_ref[...],
                             preferred_element_type=jnp.float32) + b2_ref[...]

    n = h.shape[0]
    br = 2000
    assert n % br == 0
    return pl.pallas_call(
        body,
        grid=(n // br,),
        in_specs=[
            pl.BlockSpec((br, 2 * D), lambda i: (i, 0)),
            pl.BlockSpec((2 * D, 2 * D), lambda i: (0, 0)),
            pl.BlockSpec((1, 2 * D), lambda i: (0, 0)),
            pl.BlockSpec((2 * D, D), lambda i: (0, 0)),
            pl.BlockSpec((1, D), lambda i: (0, 0)),
            pl.BlockSpec((D, 1), lambda i: (0, 0)),
            pl.BlockSpec((1, 1), lambda i: (0, 0)),
        ],
        out_specs=pl.BlockSpec((br, 1), lambda i: (i, 0)),
        out_shape=jax.ShapeDtypeStruct((n, 1), jnp.float32),
    )(h, w0t, b0, w1t, b1, w2t, b2)


def kernel(x_1st, x_2nd, edge, A_causal, A_trivial,
           W_node_0, b_node_0, W_node_1, b_node_1,
           W_r0, b_r0, W_r1, b_r1, W_r2, b_r2):
    xc = jnp.transpose(x_1st[0], (1, 0, 2))
    x0 = jnp.pad(xc, ((0, 0), (0, TP - T), (0, 0))).reshape(2 * TP, D)
    pad = ((0, 0), (0, EPTP - EPT))
    nin_p = jnp.pad(edge[0].reshape(NS, EPT), pad)
    nout_p = jnp.pad(edge[1].reshape(NS, EPT), pad)
    pk0 = jnp.bitwise_or(nout_p, jnp.left_shift(nin_p, 15))
    pk1 = jnp.bitwise_or(nout_p + TP, jnp.left_shift(nin_p, 15))
    packed = jnp.concatenate([pk0, pk1]).reshape(2 * NS * NCHUNK, K)
    a = jnp.pad(A_causal.reshape(NS, EPT), pad).reshape(NS * NCHUNK, K)

    m0 = _sc_message_passing(x0, packed, a)
    x1 = _tc_linear(m0, W_node_0.T, b_node_0.reshape(1, D), relu=True)
    m1 = _sc_message_passing(x1, packed, a)
    x2 = _tc_linear(m1, W_node_1.T, b_node_1.reshape(1, D), relu=False)

    h = jnp.concatenate([x2[:T], x2[TP:TP + T]], axis=1)
    o = _tc_readout(h, W_r0.T, b_r0.reshape(1, 2 * D),
                    W_r1.T, b_r1.reshape(1, D),
                    W_r2.T, b_r2.reshape(1, 1))
    return o.reshape(1, T, 1)


# speedup vs baseline: 5.2216x; 1.0002x over previous
import functools

import jax
import jax.numpy as jnp
from jax import lax
from jax.experimental import pallas as pl
from jax.experimental.pallas import tpu as pltpu
from jax.experimental.pallas import tpu_sc as plsc

T = 10000
TP = 10240
M = 160000
D = 128
NS = 16
EPT = M // NS
K = 64
EPTP = 10240
NCHUNK = EPTP // K
NBUF = 3
RPT = TP // NS


def _sc_message_passing(x_cat, packed, a):
    mesh = plsc.VectorSubcoreMesh(core_axis_name="c", subcore_axis_name="s")

    @functools.partial(
        pl.kernel,
        mesh=mesh,
        out_type=jax.ShapeDtypeStruct((2 * TP, D), jnp.float32),
        scratch_types=[
            pltpu.VMEM_SHARED((TP, D), jnp.float32),
            pltpu.VMEM((NCHUNK, K), jnp.int32),
            pltpu.VMEM((NBUF, K, D), jnp.float32),
            pltpu.VMEM((NBUF, K), jnp.float32),
            pltpu.VMEM((NBUF, K), jnp.int32),
            pltpu.VMEM((NBUF, K), jnp.int32),
            pltpu.SemaphoreType.DMA((NBUF,)),
            pltpu.SemaphoreType.DMA((NBUF,)),
        ],
    )
    def k(x_hbm, packed_hbm, a_hbm, out_hbm,
          acc, pk_v, rowsb, abb, gib, sib, gsem, ssem):
        c = lax.axis_index("c")
        s = lax.axis_index("s")
        rows = tuple(rowsb.at[b] for b in range(NBUF))
        ab = tuple(abb.at[b] for b in range(NBUF))
        gidx = tuple(gib.at[b] for b in range(NBUF))
        sidx = tuple(sib.at[b] for b in range(NBUF))

        pltpu.sync_copy(packed_hbm.at[pl.ds((c * NS + s) * NCHUNK, NCHUNK)],
                        pk_v)
        pltpu.sync_copy(x_hbm.at[pl.ds(c * TP + s * RPT, RPT)],
                        acc.at[pl.ds(s * RPT, RPT)])

        def unpack_idx(i, b):
            for v in range(K // 16):
                sl = pl.ds(v * 16, 16)
                pk = pk_v[i, sl]
                gidx[b][sl] = jnp.bitwise_and(pk, 0x7FFF)
                sidx[b][sl] = jnp.right_shift(pk, 15)

        def mul_rows(b):
            def emul(g, carry):
                ae16 = ab[b][pl.ds(g * 16, 16)]
                for j in range(16):
                    e = g * 16 + j
                    aj = jnp.full((16,), ae16[j], jnp.float32)
                    for dd in range(D // 16):
                        sl = pl.ds(dd * 16, 16)
                        rows[b][e, sl] = rows[b][e, sl] * aj
                return carry

            lax.fori_loop(0, K // 16, emul, 0)

        def gather_start(i, b):
            pltpu.async_copy(x_hbm.at[gidx[b]], rows[b], gsem.at[b])
            pltpu.async_copy(a_hbm.at[s * NCHUNK + i], ab[b], gsem.at[b])

        def gather_wait(i, b):
            pltpu.make_async_copy(x_hbm.at[gidx[b]], rows[b],
                                  gsem.at[b]).wait()
            pltpu.make_async_copy(a_hbm.at[s * NCHUNK + i], ab[b],
                                  gsem.at[b]).wait()

        def scatter_start(b):
            pltpu.async_copy(rows[b], acc.at[sidx[b]], ssem.at[b], add=True)

        def scatter_wait(b):
            pltpu.make_async_copy(rows[b], acc.at[sidx[b]], ssem.at[b]).wait()

        plsc.subcore_barrier()

        def step(i, ph, first=False, gstart=True):
            b, bg = ph % NBUF, (ph + NBUF - 1) % NBUF
            gather_wait(i, b)
            if not first:
                scatter_wait(bg)
            if gstart:
                unpack_idx(i + NBUF - 1, bg)
                gather_start(i + NBUF - 1, bg)
            mul_rows(b)
            scatter_start(b)

        for b in range(NBUF - 1):
            unpack_idx(b, b)
            gather_start(b, b)
        step(0, 0, first=True)

        def pipe_body(g, carry):
            for bb in range(NBUF):
                step(NBUF * g + 1 + bb, (1 + bb) % NBUF)
            return carry

        lax.fori_loop(0, (NCHUNK - NBUF) // NBUF, pipe_body, 0)

        for i in range(NBUF * ((NCHUNK - NBUF) // NBUF) + 1, NCHUNK):
            step(i, i % NBUF, gstart=(i + NBUF - 1 < NCHUNK))
        scatter_wait((NCHUNK - 1) % NBUF)

        plsc.subcore_barrier()
        pltpu.sync_copy(acc.at[pl.ds(s * RPT, RPT)],
                        out_hbm.at[pl.ds(c * TP + s * RPT, RPT)])

    return k(x_cat, packed, a)


def _tc_linear(x, wt, b, relu):
    def body(x_ref, w_ref, b_ref, o_ref):
        y = jnp.dot(x_ref[...], w_ref[...],
                    preferred_element_type=jnp.float32) + b_ref[...]
        if relu:
            y = jnp.maximum(y, 0.0)
        o_ref[...] = y

    n, din = x.shape
    dout = wt.shape[1]
    br = 2048
    assert n % br == 0
    return pl.pallas_call(
        body,
        grid=(n // br,),
        in_specs=[
            pl.BlockSpec((br, din), lambda i: (i, 0)),
            pl.BlockSpec((din, dout), lambda i: (0, 0)),
            pl.BlockSpec((1, dout), lambda i: (0, 0)),
        ],
        out_specs=pl.BlockSpec((br, dout), lambda i: (i, 0)),
        out_shape=jax.ShapeDtypeStruct((n, dout), jnp.float32),
    )(x, wt, b)


def _tc_readout(h, w0t, b0, w1t, b1, w2t, b2):
    def body(h_ref, w0_ref, b0_ref, w1_ref, b1_ref, w2_ref, b2_ref, o_ref):
        hh = jnp.dot(h_ref[...], w0_ref[...],
                     preferred_element_type=jnp.float32) + b0_ref[...]
        hh = jnp.maximum(hh, 0.0)
        hh = jnp.dot(hh, w1_ref[...],
                     preferred_element_type=jnp.float32) + b1_ref[...]
        hh = jnp.maximum(hh, 0.0)
        o_ref[...] = jnp.dot(hh, w2_ref[...],
                             preferred_element_type=jnp.float32) + b2_ref[...]

    n = h.shape[0]
    br = 2000
    assert n % br == 0
    return pl.pallas_call(
        body,
        grid=(n // br,),
        in_specs=[
            pl.BlockSpec((br, 2 * D), lambda i: (i, 0)),
            pl.BlockSpec((2 * D, 2 * D), lambda i: (0, 0)),
            pl.BlockSpec((1, 2 * D), lambda i: (0, 0)),
            pl.BlockSpec((2 * D, D), lambda i: (0, 0)),
            pl.BlockSpec((1, D), lambda i: (0, 0)),
            pl.BlockSpec((D, 1), lambda i: (0, 0)),
            pl.BlockSpec((1, 1), lambda i: (0, 0)),
        ],
        out_specs=pl.BlockSpec((br, 1), lambda i: (i, 0)),
        out_shape=jax.ShapeDtypeStruct((n, 1), jnp.float32),
    )(h, w0t, b0, w1t, b1, w2t, b2)


def kernel(x_1st, x_2nd, edge, A_causal, A_trivial,
           W_node_0, b_node_0, W_node_1, b_node_1,
           W_r0, b_r0, W_r1, b_r1, W_r2, b_r2):
    xc = jnp.transpose(x_1st[0], (1, 0, 2))
    x0 = jnp.pad(xc, ((0, 0), (0, TP - T), (0, 0))).reshape(2 * TP, D)
    pad = ((0, 0), (0, EPTP - EPT))
    nin_p = jnp.pad(edge[0].reshape(NS, EPT), pad)
    nout_p = jnp.pad(edge[1].reshape(NS, EPT), pad)
    pk0 = jnp.bitwise_or(nout_p, jnp.left_shift(nin_p, 15))
    pk1 = jnp.bitwise_or(nout_p + TP, jnp.left_shift(nin_p, 15))
    packed = jnp.concatenate([pk0, pk1]).reshape(2 * NS * NCHUNK, K)
    a = jnp.pad(A_causal.reshape(NS, EPT), pad).reshape(NS * NCHUNK, K)

    m0 = _sc_message_passing(x0, packed, a)
    x1 = _tc_linear(m0, W_node_0.T, b_node_0.reshape(1, D), relu=True)
    m1 = _sc_message_passing(x1, packed, a)
    x2 = _tc_linear(m1, W_node_1.T, b_node_1.reshape(1, D), relu=False)

    h = jnp.concatenate([x2[:T], x2[TP:TP + T]], axis=1)
    o = _tc_readout(h, W_r0.T, b_r0.reshape(1, 2 * D),
                    W_r1.T, b_r1.reshape(1, D),
                    W_r2.T, b_r2.reshape(1, 1))
    return o.reshape(1, T, 1)


# speedup vs baseline: 5.3659x; 1.0277x over previous
import functools

import jax
import jax.numpy as jnp
from jax import lax
from jax.experimental import pallas as pl
from jax.experimental.pallas import tpu as pltpu
from jax.experimental.pallas import tpu_sc as plsc

T = 10000
TP = 10240
M = 160000
D = 128
NS = 16
EPT = M // NS
K = 64
EPTP = 10240
NCHUNK = EPTP // K
NBUF = 3
RPT = TP // NS


def _sc_message_passing(x_cat, packed, a):
    mesh = plsc.VectorSubcoreMesh(core_axis_name="c", subcore_axis_name="s")

    @functools.partial(
        pl.kernel,
        mesh=mesh,
        out_type=jax.ShapeDtypeStruct((2 * TP, D), jnp.float32),
        scratch_types=[
            pltpu.VMEM_SHARED((TP, D), jnp.float32),
            pltpu.VMEM((NCHUNK, K), jnp.int32),
            pltpu.VMEM((NBUF, K, D), jnp.float32),
            pltpu.VMEM((NBUF, K), jnp.float32),
            pltpu.VMEM((NBUF, K), jnp.int32),
            pltpu.VMEM((NBUF, K), jnp.int32),
            pltpu.SemaphoreType.DMA((NBUF,)),
            pltpu.SemaphoreType.DMA((NBUF,)),
        ],
    )
    def k(x_hbm, packed_hbm, a_hbm, out_hbm,
          acc, pk_v, rowsb, abb, gib, sib, gsem, ssem):
        c = lax.axis_index("c")
        s = lax.axis_index("s")
        rows = tuple(rowsb.at[b] for b in range(NBUF))
        ab = tuple(abb.at[b] for b in range(NBUF))
        gidx = tuple(gib.at[b] for b in range(NBUF))
        sidx = tuple(sib.at[b] for b in range(NBUF))

        pltpu.sync_copy(packed_hbm.at[pl.ds((c * NS + s) * NCHUNK, NCHUNK)],
                        pk_v)
        pltpu.sync_copy(x_hbm.at[pl.ds(c * TP + s * RPT, RPT)],
                        acc.at[pl.ds(s * RPT, RPT)])

        def unpack_idx(i, b):
            for v in range(K // 16):
                sl = pl.ds(v * 16, 16)
                pk = pk_v[i, sl]
                gidx[b][sl] = jnp.bitwise_and(pk, 0x7FFF)
                sidx[b][sl] = jnp.right_shift(pk, 15)

        def mul_rows(b):
            def emul(g, carry):
                ae16 = ab[b][pl.ds(g * 16, 16)]
                for j in range(16):
                    e = g * 16 + j
                    aj = jnp.full((16,), ae16[j], jnp.float32)
                    for dd in range(D // 16):
                        sl = pl.ds(dd * 16, 16)
                        rows[b][e, sl] = rows[b][e, sl] * aj
                return carry

            lax.fori_loop(0, K // 16, emul, 0)

        def gather_start(i, b):
            pltpu.async_copy(x_hbm.at[gidx[b]], rows[b], gsem.at[b])
            pltpu.async_copy(a_hbm.at[s * NCHUNK + i], ab[b], gsem.at[b])

        def gather_wait(i, b):
            pltpu.make_async_copy(x_hbm.at[gidx[b]], rows[b],
                                  gsem.at[b]).wait()
            pltpu.make_async_copy(a_hbm.at[s * NCHUNK + i], ab[b],
                                  gsem.at[b]).wait()

        def scatter_start(b):
            pltpu.async_copy(rows[b], acc.at[sidx[b]], ssem.at[b], add=True)

        def scatter_wait(b):
            pltpu.make_async_copy(rows[b], acc.at[sidx[b]], ssem.at[b]).wait()

        plsc.subcore_barrier()

        def step(i, ph, first=False, gstart=True):
            b, bg = ph % NBUF, (ph + NBUF - 1) % NBUF
            gather_wait(i, b)
            if not first:
                scatter_wait(bg)
            if gstart:
                unpack_idx(i + NBUF - 1, bg)
                gather_start(i + NBUF - 1, bg)
            mul_rows(b)
            scatter_start(b)

        for b in range(NBUF - 1):
            unpack_idx(b, b)
            gather_start(b, b)
        step(0, 0, first=True)

        def pipe_body(g, carry):
            for bb in range(NBUF):
                step(NBUF * g + 1 + bb, (1 + bb) % NBUF)
            return carry

        lax.fori_loop(0, (NCHUNK - NBUF) // NBUF, pipe_body, 0)

        for i in range(NBUF * ((NCHUNK - NBUF) // NBUF) + 1, NCHUNK):
            step(i, i % NBUF, gstart=(i + NBUF - 1 < NCHUNK))
        scatter_wait((NCHUNK - 1) % NBUF)

        plsc.subcore_barrier()
        pltpu.sync_copy(acc.at[pl.ds(s * RPT, RPT)],
                        out_hbm.at[pl.ds(c * TP + s * RPT, RPT)])

    return k(x_cat, packed, a)


def _tc_linear(x, wt, b, relu):
    def body(x_ref, w_ref, b_ref, o_ref):
        y = jnp.dot(x_ref[...], w_ref[...],
                    preferred_element_type=jnp.float32) + b_ref[...]
        if relu:
            y = jnp.maximum(y, 0.0)
        o_ref[...] = y

    n, din = x.shape
    dout = wt.shape[1]
    br = 2048
    assert n % br == 0
    return pl.pallas_call(
        body,
        grid=(n // br,),
        in_specs=[
            pl.BlockSpec((br, din), lambda i: (i, 0)),
            pl.BlockSpec((din, dout), lambda i: (0, 0)),
            pl.BlockSpec((1, dout), lambda i: (0, 0)),
        ],
        out_specs=pl.BlockSpec((br, dout), lambda i: (i, 0)),
        out_shape=jax.ShapeDtypeStruct((n, dout), jnp.float32),
    )(x, wt, b)


def _tc_final(m1, wnt, bn, w0t, b0, w1t, b1, w2t, b2):
    def body(x0_ref, x1_ref, wn_ref, bn_ref, w0_ref, b0_ref,
             w1_ref, b1_ref, w2_ref, b2_ref, o_ref):
        y0 = jnp.dot(x0_ref[...], wn_ref[...],
                     preferred_element_type=jnp.float32) + bn_ref[...]
        y1 = jnp.dot(x1_ref[...], wn_ref[...],
                     preferred_element_type=jnp.float32) + bn_ref[...]
        hh = jnp.concatenate([y0, y1], axis=1)
        hh = jnp.maximum(jnp.dot(hh, w0_ref[...],
                                 preferred_element_type=jnp.float32)
                         + b0_ref[...], 0.0)
        hh = jnp.maximum(jnp.dot(hh, w1_ref[...],
                                 preferred_element_type=jnp.float32)
                         + b1_ref[...], 0.0)
        o_ref[...] = jnp.dot(hh, w2_ref[...],
                             preferred_element_type=jnp.float32) + b2_ref[...]

    br = 1024
    nblk = TP // br
    return pl.pallas_call(
        body,
        grid=(nblk,),
        in_specs=[
            pl.BlockSpec((br, D), lambda i: (i, 0)),
            pl.BlockSpec((br, D), lambda i: (nblk + i, 0)),
            pl.BlockSpec((D, D), lambda i: (0, 0)),
            pl.BlockSpec((1, D), lambda i: (0, 0)),
            pl.BlockSpec((2 * D, 2 * D), lambda i: (0, 0)),
            pl.BlockSpec((1, 2 * D), lambda i: (0, 0)),
            pl.BlockSpec((2 * D, D), lambda i: (0, 0)),
            pl.BlockSpec((1, D), lambda i: (0, 0)),
            pl.BlockSpec((D, 1), lambda i: (0, 0)),
            pl.BlockSpec((1, 1), lambda i: (0, 0)),
        ],
        out_specs=pl.BlockSpec((br, 1), lambda i: (i, 0)),
        out_shape=jax.ShapeDtypeStruct((TP, 1), jnp.float32),
    )(m1, m1, wnt, bn, w0t, b0, w1t, b1, w2t, b2)


def kernel(x_1st, x_2nd, edge, A_causal, A_trivial,
           W_node_0, b_node_0, W_node_1, b_node_1,
           W_r0, b_r0, W_r1, b_r1, W_r2, b_r2):
    xc = jnp.transpose(x_1st[0], (1, 0, 2))
    x0 = jnp.pad(xc, ((0, 0), (0, TP - T), (0, 0))).reshape(2 * TP, D)
    pad = ((0, 0), (0, EPTP - EPT))
    nin_p = jnp.pad(edge[0].reshape(NS, EPT), pad)
    nout_p = jnp.pad(edge[1].reshape(NS, EPT), pad)
    pk0 = jnp.bitwise_or(nout_p, jnp.left_shift(nin_p, 15))
    pk1 = jnp.bitwise_or(nout_p + TP, jnp.left_shift(nin_p, 15))
    packed = jnp.concatenate([pk0, pk1]).reshape(2 * NS * NCHUNK, K)
    a = jnp.pad(A_causal.reshape(NS, EPT), pad).reshape(NS * NCHUNK, K)

    m0 = _sc_message_passing(x0, packed, a)
    x1 = _tc_linear(m0, W_node_0.T, b_node_0.reshape(1, D), relu=True)
    m1 = _sc_message_passing(x1, packed, a)
    o = _tc_final(m1, W_node_1.T, b_node_1.reshape(1, D),
                  W_r0.T, b_r0.reshape(1, 2 * D),
                  W_r1.T, b_r1.reshape(1, D),
                  W_r2.T, b_r2.reshape(1, 1))
    return o[:T].reshape(1, T, 1)
